# baseline (speedup 1.0000x reference)
import functools

import jax
import jax.numpy as jnp
from jax import lax
from jax.experimental import pallas as pl
from jax.experimental.pallas import tpu as pltpu
from jax.experimental.pallas import tpu_sc as plsc

N = 10000
E = 320000
D = 128

NC = 2
NS = 16
NW = NC * NS
C = 128
TCH = 2560
FCH = 104
SCH = 56
TCH_PAD = 2624
HALF = 64
NB = 2
E_PAD = TCH * C
DUMP = N
RPT = 632
NROWS = NS * RPT
NCNT = 10048

_mesh = plsc.VectorSubcoreMesh(core_axis_name="c", subcore_axis_name="s")


def _make_agg(with_count):
  out_type = [jax.ShapeDtypeStruct((2 * NROWS, D), jnp.float32)]
  scratch = [
      pltpu.VMEM((HALF, C), jnp.int32),
      pltpu.VMEM((NB, C), jnp.int32),
      pltpu.VMEM((NB, C), jnp.int32),
      pltpu.VMEM((NB, C, D), jnp.float32),
      pltpu.VMEM_SHARED((NROWS, D), jnp.float32),
      pltpu.SemaphoreType.DMA,
  ]
  if with_count:
    out_type.append(jax.ShapeDtypeStruct((2 * NCNT,), jnp.float32))
    scratch += [
        pltpu.VMEM((C,), jnp.float32),
        pltpu.VMEM((640,), jnp.float32),
        pltpu.VMEM_SHARED((NCNT,), jnp.float32),
    ]

  def body(h_hbm, pk_hbm, *refs):
    if with_count:
      (out_hbm, cnt_hbm, pk_v, sidx_v, didx_v, rows_v, acc_sh, gsem,
       ones_v, z1_v, cnt_sh) = refs
    else:
      out_hbm, pk_v, sidx_v, didx_v, rows_v, acc_sh, gsem = refs

    c = lax.axis_index("c")
    s = lax.axis_index("s")
    r0 = s * RPT

    nch = jnp.where(c == 0, FCH, SCH)
    base = jnp.where(c == 0, s * FCH, NS * FCH + s * SCH)

    @pl.when(c == 0)
    def _():
      pltpu.sync_copy(pk_hbm.at[pl.ds(base, HALF)], pk_v)

    if SCH > 0:
      @pl.when(c == 1)
      def _():
        pltpu.sync_copy(pk_hbm.at[pl.ds(base, SCH)], pk_v.at[pl.ds(0, SCH)])

    cN = c * N

    def unpack(jj, b):
      for i in range(C // 16):
        p = pk_v[jj, pl.ds(16 * i, 16)]
        sidx_v[b, pl.ds(16 * i, 16)] = lax.bitwise_and(p, 0xFFFF) + cN
        didx_v[b, pl.ds(16 * i, 16)] = lax.shift_right_logical(p, 16)

    def zrow(i, _):
      for k in range(D // 16):
        rows_v[0, i, pl.ds(16 * k, 16)] = jnp.zeros((16,), jnp.float32)
      return 0
    lax.fori_loop(0, C, zrow, 0)
    for k in range(4):
      pltpu.sync_copy(rows_v.at[0], acc_sh.at[pl.ds(r0 + C * k, C)])
    pltpu.sync_copy(rows_v.at[0, pl.ds(0, RPT - 4 * C)],
                    acc_sh.at[pl.ds(r0 + 4 * C, RPT - 4 * C)])

    if with_count:
      def zinit(i, _):
        ones_v[pl.ds(16 * i, 16)] = jnp.ones((16,), jnp.float32)
        return 0
      lax.fori_loop(0, C // 16, zinit, 0)

      def zinit2(i, _):
        z1_v[pl.ds(16 * i, 16)] = jnp.zeros((16,), jnp.float32)
        return 0
      lax.fori_loop(0, 640 // 16, zinit2, 0)

      @pl.when(s < NS - 1)
      def _():
        pltpu.sync_copy(z1_v, cnt_sh.at[pl.ds(640 * s, 640)])

      @pl.when(s == NS - 1)
      def _():
        pltpu.sync_copy(z1_v.at[pl.ds(0, 448)],
                        cnt_sh.at[pl.ds(9600, 448)])

    plsc.subcore_barrier()

    def g_wait():
      pltpu.make_async_copy(h_hbm.at[pl.ds(0, C)], rows_v.at[0], gsem).wait()

    @pl.when(nch > 0)
    def _():
      unpack(0, 0)
      pltpu.async_copy(h_hbm.at[sidx_v.at[0]], rows_v.at[0], gsem)

    def step(j, _):
      b = lax.rem(j, NB)
      b1 = lax.rem(j + 1, NB)

      @pl.when((lax.rem(j + 1, HALF) == 0) & (j + 1 < nch))
      def _():
        off = pl.multiple_of(base + j + 1, 8)
        pltpu.sync_copy(pk_hbm.at[pl.ds(off, HALF)], pk_v)

      @pl.when(j + 1 < nch)
      def _():
        unpack(lax.rem(j + 1, HALF), b1)
        pltpu.async_copy(h_hbm.at[sidx_v.at[b1]], rows_v.at[b1], gsem)

      g_wait()
      pltpu.sync_copy(rows_v.at[b], acc_sh.at[didx_v.at[b]], add=True)
      if with_count:
        pltpu.sync_copy(ones_v, cnt_sh.at[didx_v.at[b]], add=True)
      return 0
    lax.fori_loop(0, nch, step, 0)

    plsc.subcore_barrier()

    pltpu.sync_copy(acc_sh.at[pl.ds(r0, RPT)],
                    out_hbm.at[pl.ds(c * NROWS + r0, RPT)])
    if with_count:
      @pl.when(s < NS - 1)
      def _():
        pltpu.sync_copy(cnt_sh.at[pl.ds(640 * s, 640)], z1_v)
        pltpu.sync_copy(z1_v, cnt_hbm.at[pl.ds(c * NCNT + 640 * s, 640)])

      @pl.when(s == NS - 1)
      def _():
        pltpu.sync_copy(cnt_sh.at[pl.ds(9600, 448)], z1_v.at[pl.ds(0, 448)])
        pltpu.sync_copy(z1_v.at[pl.ds(0, 448)],
                        cnt_hbm.at[pl.ds(c * NCNT + 9600, 448)])

  return pl.kernel(body, out_type=out_type, mesh=_mesh,
                   scratch_types=scratch)


_agg_cnt = _make_agg(True)
_agg = _make_agg(False)


BN = 400


def _combine_body(relu, dup, p_ref, cnt_ref, h_ref, wl_ref, wr_ref, b_ref,
                  o_ref):
  cnt = cnt_ref[0, :, 0] + cnt_ref[1, :, 0]
  cnt = jnp.maximum(cnt, 1.0)
  agg = (p_ref[0] + p_ref[1]) / cnt[:, None]
  z = (lax.dot_general(agg, wl_ref[...], (((1,), (0,)), ((), ())),
                       precision=lax.Precision.HIGHEST,
                       preferred_element_type=jnp.float32)
       + lax.dot_general(h_ref[...], wr_ref[...], (((1,), (0,)), ((), ())),
                         precision=lax.Precision.HIGHEST,
                         preferred_element_type=jnp.float32)
       + b_ref[...])
  z = jnp.maximum(z, 0.0) if relu else z
  if dup:
    o_ref[...] = jnp.broadcast_to(z[None], (2, z.shape[0], z.shape[1]))
  else:
    o_ref[...] = z


def _combine(p, cnt, h, wl, wr, b, relu, dup=True):
  if dup:
    out_spec = pl.BlockSpec((2, BN, D), lambda i: (0, i, 0))
    out_shape = jax.ShapeDtypeStruct((2, N, D), jnp.float32)
  else:
    out_spec = pl.BlockSpec((BN, D), lambda i: (i, 0))
    out_shape = jax.ShapeDtypeStruct((N, D), jnp.float32)
  f = pl.pallas_call(
      functools.partial(_combine_body, relu, dup),
      grid=(N // BN,),
      in_specs=[
          pl.BlockSpec((2, BN, D), lambda i: (0, i, 0)),
          pl.BlockSpec((2, BN, 1), lambda i: (0, i, 0)),
          pl.BlockSpec((BN, D), lambda i: (i, 0)),
          pl.BlockSpec((D, D), lambda i: (0, 0)),
          pl.BlockSpec((D, D), lambda i: (0, 0)),
          pl.BlockSpec((1, D), lambda i: (0, 0)),
      ],
      out_specs=out_spec,
      out_shape=out_shape,
  )
  return f(p, cnt, h, wl, wr, b)


def kernel(x, edge_index, W_l0, b_l0, W_r0, W_l1, b_l1, W_r1,
           W_l2, b_l2, W_r2):
  src = edge_index[0]
  dst = edge_index[1]
  pad = E_PAD - E
  packed = jnp.bitwise_or(jnp.left_shift(dst, 16), src)
  packed = jnp.concatenate(
      [packed, jnp.full((pad,), DUMP << 16, jnp.int32)]).reshape(TCH, C)
  packed = jnp.pad(packed, ((0, TCH_PAD - TCH), (0, 0)))

  xdup = jnp.concatenate([x, x], axis=0)
  p0, cflat = _agg_cnt(xdup, packed)
  cnt = cflat.reshape(2, NCNT, 1)
  h1 = _combine(p0.reshape(2, NROWS, D), cnt, x, W_l0, W_r0,
                b_l0.reshape(1, D), relu=True)
  h1f = h1.reshape(2 * N, D)
  (p1,) = _agg(h1f, packed)
  h2 = _combine(p1.reshape(2, NROWS, D), cnt, h1f, W_l1, W_r1,
                b_l1.reshape(1, D), relu=True)
  h2f = h2.reshape(2 * N, D)
  (p2,) = _agg(h2f, packed)
  out = _combine(p2.reshape(2, NROWS, D), cnt, h2f, W_l2, W_r2,
                 b_l2.reshape(1, D), relu=False, dup=False)
  return out


# speedup vs baseline: 1.0063x; 1.0063x over previous
import functools

import jax
import jax.numpy as jnp
from jax import lax
from jax.experimental import pallas as pl
from jax.experimental.pallas import tpu as pltpu
from jax.experimental.pallas import tpu_sc as plsc

N = 10000
E = 320000
D = 128

NC = 2
NS = 16
NW = NC * NS
C = 128
TCH = 2560
FCH = 128
SCH = 32
TCH_PAD = 2624
HALF = 64
NB = 2
E_PAD = TCH * C
DUMP = N
RPT = 632
NROWS = NS * RPT
NCNT = 10048

_mesh = plsc.VectorSubcoreMesh(core_axis_name="c", subcore_axis_name="s")


def _make_agg(with_count):
  out_type = [jax.ShapeDtypeStruct((2 * NROWS, D), jnp.float32)]
  scratch = [
      pltpu.VMEM((HALF, C), jnp.int32),
      pltpu.VMEM((NB, C), jnp.int32),
      pltpu.VMEM((NB, C), jnp.int32),
      pltpu.VMEM((NB, C, D), jnp.float32),
      pltpu.VMEM_SHARED((NROWS, D), jnp.float32),
      pltpu.SemaphoreType.DMA,
      pltpu.SemaphoreType.DMA,
  ]
  if with_count:
    out_type.append(jax.ShapeDtypeStruct((2 * NCNT,), jnp.float32))
    scratch += [
        pltpu.VMEM((C,), jnp.float32),
        pltpu.VMEM((640,), jnp.float32),
        pltpu.VMEM_SHARED((NCNT,), jnp.float32),
    ]

  def body(h_hbm, pk_hbm, *refs):
    if with_count:
      (out_hbm, cnt_hbm, pk_v, sidx_v, didx_v, rows_v, acc_sh, gsem, ssem,
       ones_v, z1_v, cnt_sh) = refs
    else:
      out_hbm, pk_v, sidx_v, didx_v, rows_v, acc_sh, gsem, ssem = refs

    c = lax.axis_index("c")
    s = lax.axis_index("s")
    r0 = s * RPT

    nch = jnp.where(c == 0, FCH, SCH)
    base = jnp.where(c == 0, s * FCH, NS * FCH + s * SCH)

    @pl.when(c == 0)
    def _():
      pltpu.sync_copy(pk_hbm.at[pl.ds(base, HALF)], pk_v)

    if SCH > 0:
      @pl.when(c == 1)
      def _():
        pltpu.sync_copy(pk_hbm.at[pl.ds(base, SCH)], pk_v.at[pl.ds(0, SCH)])

    cN = c * N

    def unpack(jj, b):
      for i in range(C // 16):
        p = pk_v[jj, pl.ds(16 * i, 16)]
        sidx_v[b, pl.ds(16 * i, 16)] = lax.bitwise_and(p, 0xFFFF) + cN
        didx_v[b, pl.ds(16 * i, 16)] = lax.shift_right_logical(p, 16)

    def zrow(i, _):
      for k in range(D // 16):
        rows_v[0, i, pl.ds(16 * k, 16)] = jnp.zeros((16,), jnp.float32)
      return 0
    lax.fori_loop(0, C, zrow, 0)
    for k in range(4):
      pltpu.sync_copy(rows_v.at[0], acc_sh.at[pl.ds(r0 + C * k, C)])
    pltpu.sync_copy(rows_v.at[0, pl.ds(0, RPT - 4 * C)],
                    acc_sh.at[pl.ds(r0 + 4 * C, RPT - 4 * C)])

    if with_count:
      def zinit(i, _):
        ones_v[pl.ds(16 * i, 16)] = jnp.ones((16,), jnp.float32)
        return 0
      lax.fori_loop(0, C // 16, zinit, 0)

      def zinit2(i, _):
        z1_v[pl.ds(16 * i, 16)] = jnp.zeros((16,), jnp.float32)
        return 0
      lax.fori_loop(0, 640 // 16, zinit2, 0)

      @pl.when(s < NS - 1)
      def _():
        pltpu.sync_copy(z1_v, cnt_sh.at[pl.ds(640 * s, 640)])

      @pl.when(s == NS - 1)
      def _():
        pltpu.sync_copy(z1_v.at[pl.ds(0, 448)],
                        cnt_sh.at[pl.ds(9600, 448)])

    plsc.subcore_barrier()

    def g_wait():
      pltpu.make_async_copy(h_hbm.at[pl.ds(0, C)], rows_v.at[0], gsem).wait()

    def s_wait():
      pltpu.make_async_copy(h_hbm.at[pl.ds(0, C)], rows_v.at[0], ssem).wait()

    @pl.when(nch > 0)
    def _():
      unpack(0, 0)
      pltpu.async_copy(h_hbm.at[sidx_v.at[0]], rows_v.at[0], gsem)

    def step(j, _):
      b = lax.rem(j, NB)
      b1 = lax.rem(j + 1, NB)

      @pl.when((lax.rem(j + 1, HALF) == 0) & (j + 1 < nch))
      def _():
        off = pl.multiple_of(base + j + 1, 8)
        pltpu.sync_copy(pk_hbm.at[pl.ds(off, HALF)], pk_v)

      @pl.when(j + 1 < nch)
      def _():
        unpack(lax.rem(j + 1, HALF), b1)

        @pl.when(j >= 1)
        def _():
          s_wait()
        pltpu.async_copy(h_hbm.at[sidx_v.at[b1]], rows_v.at[b1], gsem)

      g_wait()
      pltpu.async_copy(rows_v.at[b], acc_sh.at[didx_v.at[b]], ssem, add=True)
      if with_count:
        pltpu.sync_copy(ones_v, cnt_sh.at[didx_v.at[b]], add=True)
      return 0
    lax.fori_loop(0, nch, step, 0)

    @pl.when(nch >= 2)
    def _():
      s_wait()

    @pl.when(nch >= 1)
    def _():
      s_wait()

    plsc.subcore_barrier()

    pltpu.sync_copy(acc_sh.at[pl.ds(r0, RPT)],
                    out_hbm.at[pl.ds(c * NROWS + r0, RPT)])
    if with_count:
      @pl.when(s < NS - 1)
      def _():
        pltpu.sync_copy(cnt_sh.at[pl.ds(640 * s, 640)], z1_v)
        pltpu.sync_copy(z1_v, cnt_hbm.at[pl.ds(c * NCNT + 640 * s, 640)])

      @pl.when(s == NS - 1)
      def _():
        pltpu.sync_copy(cnt_sh.at[pl.ds(9600, 448)], z1_v.at[pl.ds(0, 448)])
        pltpu.sync_copy(z1_v.at[pl.ds(0, 448)],
                        cnt_hbm.at[pl.ds(c * NCNT + 9600, 448)])

  return pl.kernel(body, out_type=out_type, mesh=_mesh,
                   scratch_types=scratch)


_agg_cnt = _make_agg(True)
_agg = _make_agg(False)


BN = 400


def _combine_body(relu, dup, p_ref, cnt_ref, h_ref, wl_ref, wr_ref, b_ref,
                  o_ref):
  cnt = cnt_ref[0, :, 0] + cnt_ref[1, :, 0]
  cnt = jnp.maximum(cnt, 1.0)
  agg = (p_ref[0] + p_ref[1]) / cnt[:, None]
  z = (lax.dot_general(agg, wl_ref[...], (((1,), (0,)), ((), ())),
                       precision=lax.Precision.HIGHEST,
                       preferred_element_type=jnp.float32)
       + lax.dot_general(h_ref[...], wr_ref[...], (((1,), (0,)), ((), ())),
                         precision=lax.Precision.HIGHEST,
                         preferred_element_type=jnp.float32)
       + b_ref[...])
  z = jnp.maximum(z, 0.0) if relu else z
  if dup:
    o_ref[...] = jnp.broadcast_to(z[None], (2, z.shape[0], z.shape[1]))
  else:
    o_ref[...] = z


def _combine(p, cnt, h, wl, wr, b, relu, dup=True):
  if dup:
    out_spec = pl.BlockSpec((2, BN, D), lambda i: (0, i, 0))
    out_shape = jax.ShapeDtypeStruct((2, N, D), jnp.float32)
  else:
    out_spec = pl.BlockSpec((BN, D), lambda i: (i, 0))
    out_shape = jax.ShapeDtypeStruct((N, D), jnp.float32)
  f = pl.pallas_call(
      functools.partial(_combine_body, relu, dup),
      grid=(N // BN,),
      in_specs=[
          pl.BlockSpec((2, BN, D), lambda i: (0, i, 0)),
          pl.BlockSpec((2, BN, 1), lambda i: (0, i, 0)),
          pl.BlockSpec((BN, D), lambda i: (i, 0)),
          pl.BlockSpec((D, D), lambda i: (0, 0)),
          pl.BlockSpec((D, D), lambda i: (0, 0)),
          pl.BlockSpec((1, D), lambda i: (0, 0)),
      ],
      out_specs=out_spec,
      out_shape=out_shape,
  )
  return f(p, cnt, h, wl, wr, b)


def kernel(x, edge_index, W_l0, b_l0, W_r0, W_l1, b_l1, W_r1,
           W_l2, b_l2, W_r2):
  src = edge_index[0]
  dst = edge_index[1]
  pad = E_PAD - E
  packed = jnp.bitwise_or(jnp.left_shift(dst, 16), src)
  packed = jnp.concatenate(
      [packed, jnp.full((pad,), DUMP << 16, jnp.int32)]).reshape(TCH, C)
  packed = jnp.pad(packed, ((0, TCH_PAD - TCH), (0, 0)))

  xdup = jnp.concatenate([x, x], axis=0)
  p0, cflat = _agg_cnt(xdup, packed)
  cnt = cflat.reshape(2, NCNT, 1)
  h1 = _combine(p0.reshape(2, NROWS, D), cnt, x, W_l0, W_r0,
                b_l0.reshape(1, D), relu=True)
  h1f = h1.reshape(2 * N, D)
  (p1,) = _agg(h1f, packed)
  h2 = _combine(p1.reshape(2, NROWS, D), cnt, h1f, W_l1, W_r1,
                b_l1.reshape(1, D), relu=True)
  h2f = h2.reshape(2 * N, D)
  (p2,) = _agg(h2f, packed)
  out = _combine(p2.reshape(2, NROWS, D), cnt, h2f, W_l2, W_r2,
                 b_l2.reshape(1, D), relu=False, dup=False)
  return out


# speedup vs baseline: 1.0066x; 1.0003x over previous
import functools

import jax
import jax.numpy as jnp
from jax import lax
from jax.experimental import pallas as pl
from jax.experimental.pallas import tpu as pltpu
from jax.experimental.pallas import tpu_sc as plsc

N = 10000
E = 320000
D = 128

NC = 2
NS = 16
NW = NC * NS
C = 128
TCH = 2560
FCH = 128
SCH = 32
TCH_PAD = 2624
HALF = 64
NB = 2
E_PAD = TCH * C
DUMP = N
RPT = 632
NROWS = NS * RPT
NCNT = 10048

_mesh = plsc.VectorSubcoreMesh(core_axis_name="c", subcore_axis_name="s")


def _make_agg(with_count):
  out_type = [jax.ShapeDtypeStruct((2 * NROWS, D), jnp.float32)]
  scratch = [
      pltpu.VMEM((HALF, C), jnp.int32),
      pltpu.VMEM((NB, C), jnp.int32),
      pltpu.VMEM((NB, C), jnp.int32),
      pltpu.VMEM((NB, C, D), jnp.float32),
      pltpu.VMEM_SHARED((NROWS, D), jnp.float32),
      pltpu.SemaphoreType.DMA,
  ]
  if with_count:
    out_type.append(jax.ShapeDtypeStruct((2 * NCNT,), jnp.float32))
    scratch += [
        pltpu.VMEM((C,), jnp.float32),
        pltpu.VMEM((640,), jnp.float32),
        pltpu.VMEM_SHARED((NCNT,), jnp.float32),
    ]

  def body(h_hbm, pk_hbm, *refs):
    if with_count:
      (out_hbm, cnt_hbm, pk_v, sidx_v, didx_v, rows_v, acc_sh, gsem,
       ones_v, z1_v, cnt_sh) = refs
    else:
      out_hbm, pk_v, sidx_v, didx_v, rows_v, acc_sh, gsem = refs

    c = lax.axis_index("c")
    s = lax.axis_index("s")
    r0 = s * RPT

    nch = jnp.where(c == 0, FCH, SCH)
    base = jnp.where(c == 0, s * FCH, NS * FCH + s * SCH)

    @pl.when(c == 0)
    def _():
      pltpu.sync_copy(pk_hbm.at[pl.ds(base, HALF)], pk_v)

    if SCH > 0:
      @pl.when(c == 1)
      def _():
        pltpu.sync_copy(pk_hbm.at[pl.ds(base, SCH)], pk_v.at[pl.ds(0, SCH)])

    cN = c * N

    def unpack(jj, b):
      for i in range(C // 16):
        p = pk_v[jj, pl.ds(16 * i, 16)]
        sidx_v[b, pl.ds(16 * i, 16)] = lax.bitwise_and(p, 0xFFFF) + cN
        didx_v[b, pl.ds(16 * i, 16)] = lax.shift_right_logical(p, 16)

    def zrow(i, _):
      for k in range(D // 16):
        rows_v[0, i, pl.ds(16 * k, 16)] = jnp.zeros((16,), jnp.float32)
      return 0
    lax.fori_loop(0, C, zrow, 0)
    for k in range(4):
      pltpu.sync_copy(rows_v.at[0], acc_sh.at[pl.ds(r0 + C * k, C)])
    pltpu.sync_copy(rows_v.at[0, pl.ds(0, RPT - 4 * C)],
                    acc_sh.at[pl.ds(r0 + 4 * C, RPT - 4 * C)])

    if with_count:
      def zinit(i, _):
        ones_v[pl.ds(16 * i, 16)] = jnp.ones((16,), jnp.float32)
        return 0
      lax.fori_loop(0, C // 16, zinit, 0)

      def zinit2(i, _):
        z1_v[pl.ds(16 * i, 16)] = jnp.zeros((16,), jnp.float32)
        return 0
      lax.fori_loop(0, 640 // 16, zinit2, 0)

      @pl.when(s < NS - 1)
      def _():
        pltpu.sync_copy(z1_v, cnt_sh.at[pl.ds(640 * s, 640)])

      @pl.when(s == NS - 1)
      def _():
        pltpu.sync_copy(z1_v.at[pl.ds(0, 448)],
                        cnt_sh.at[pl.ds(9600, 448)])

    plsc.subcore_barrier()

    def g_wait():
      pltpu.make_async_copy(h_hbm.at[pl.ds(0, C)], rows_v.at[0], gsem).wait()

    @pl.when(nch > 0)
    def _():
      unpack(0, 0)
      pltpu.async_copy(h_hbm.at[sidx_v.at[0]], rows_v.at[0], gsem)

    def step(j, _):
      b = lax.rem(j, NB)
      b1 = lax.rem(j + 1, NB)

      @pl.when((lax.rem(j + 1, HALF) == 0) & (j + 1 < nch))
      def _():
        off = pl.multiple_of(base + j + 1, 8)
        pltpu.sync_copy(pk_hbm.at[pl.ds(off, HALF)], pk_v)

      @pl.when(j + 1 < nch)
      def _():
        unpack(lax.rem(j + 1, HALF), b1)
        pltpu.async_copy(h_hbm.at[sidx_v.at[b1]], rows_v.at[b1], gsem)

      g_wait()
      pltpu.sync_copy(rows_v.at[b], acc_sh.at[didx_v.at[b]], add=True)
      if with_count:
        pltpu.sync_copy(ones_v, cnt_sh.at[didx_v.at[b]], add=True)
      return 0
    lax.fori_loop(0, nch, step, 0)

    plsc.subcore_barrier()

    pltpu.sync_copy(acc_sh.at[pl.ds(r0, RPT)],
                    out_hbm.at[pl.ds(c * NROWS + r0, RPT)])
    if with_count:
      @pl.when(s < NS - 1)
      def _():
        pltpu.sync_copy(cnt_sh.at[pl.ds(640 * s, 640)], z1_v)
        pltpu.sync_copy(z1_v, cnt_hbm.at[pl.ds(c * NCNT + 640 * s, 640)])

      @pl.when(s == NS - 1)
      def _():
        pltpu.sync_copy(cnt_sh.at[pl.ds(9600, 448)], z1_v.at[pl.ds(0, 448)])
        pltpu.sync_copy(z1_v.at[pl.ds(0, 448)],
                        cnt_hbm.at[pl.ds(c * NCNT + 9600, 448)])

  return pl.kernel(body, out_type=out_type, mesh=_mesh,
                   scratch_types=scratch)


_agg_cnt = _make_agg(True)
_agg = _make_agg(False)


BN = 400


def _combine_body(relu, dup, p_ref, cnt_ref, h_ref, wl_ref, wr_ref, b_ref,
                  o_ref):
  cnt = cnt_ref[0, :, 0] + cnt_ref[1, :, 0]
  cnt = jnp.maximum(cnt, 1.0)
  agg = (p_ref[0] + p_ref[1]) / cnt[:, None]
  z = (lax.dot_general(agg, wl_ref[...], (((1,), (0,)), ((), ())),
                       precision=lax.Precision.HIGHEST,
                       preferred_element_type=jnp.float32)
       + lax.dot_general(h_ref[...], wr_ref[...], (((1,), (0,)), ((), ())),
                         precision=lax.Precision.HIGHEST,
                         preferred_element_type=jnp.float32)
       + b_ref[...])
  z = jnp.maximum(z, 0.0) if relu else z
  if dup:
    o_ref[...] = jnp.broadcast_to(z[None], (2, z.shape[0], z.shape[1]))
  else:
    o_ref[...] = z


def _combine(p, cnt, h, wl, wr, b, relu, dup=True):
  if dup:
    out_spec = pl.BlockSpec((2, BN, D), lambda i: (0, i, 0))
    out_shape = jax.ShapeDtypeStruct((2, N, D), jnp.float32)
  else:
    out_spec = pl.BlockSpec((BN, D), lambda i: (i, 0))
    out_shape = jax.ShapeDtypeStruct((N, D), jnp.float32)
  f = pl.pallas_call(
      functools.partial(_combine_body, relu, dup),
      grid=(N // BN,),
      in_specs=[
          pl.BlockSpec((2, BN, D), lambda i: (0, i, 0)),
          pl.BlockSpec((2, BN, 1), lambda i: (0, i, 0)),
          pl.BlockSpec((BN, D), lambda i: (i, 0)),
          pl.BlockSpec((D, D), lambda i: (0, 0)),
          pl.BlockSpec((D, D), lambda i: (0, 0)),
          pl.BlockSpec((1, D), lambda i: (0, 0)),
      ],
      out_specs=out_spec,
      out_shape=out_shape,
  )
  return f(p, cnt, h, wl, wr, b)


def kernel(x, edge_index, W_l0, b_l0, W_r0, W_l1, b_l1, W_r1,
           W_l2, b_l2, W_r2):
  src = edge_index[0]
  dst = edge_index[1]
  pad = E_PAD - E
  packed = jnp.bitwise_or(jnp.left_shift(dst, 16), src)
  packed = jnp.concatenate(
      [packed, jnp.full((pad,), DUMP << 16, jnp.int32)]).reshape(TCH, C)
  packed = jnp.pad(packed, ((0, TCH_PAD - TCH), (0, 0)))

  xdup = jnp.concatenate([x, x], axis=0)
  p0, cflat = _agg_cnt(xdup, packed)
  cnt = cflat.reshape(2, NCNT, 1)
  h1 = _combine(p0.reshape(2, NROWS, D), cnt, x, W_l0, W_r0,
                b_l0.reshape(1, D), relu=True)
  h1f = h1.reshape(2 * N, D)
  (p1,) = _agg(h1f, packed)
  h2 = _combine(p1.reshape(2, NROWS, D), cnt, h1f, W_l1, W_r1,
                b_l1.reshape(1, D), relu=True)
  h2f = h2.reshape(2 * N, D)
  (p2,) = _agg(h2f, packed)
  out = _combine(p2.reshape(2, NROWS, D), cnt, h2f, W_l2, W_r2,
                 b_l2.reshape(1, D), relu=False, dup=False)
  return out
